# baseline (speedup 1.0000x reference)
import jax
import jax.numpy as jnp
from jax.experimental import pallas as pl
from jax.experimental.pallas import tpu as pltpu

E = 8
K = 2
D = 1024
F = 4096
T = 2048
TM = 128
TN = 512
P = T * K + E * TM
NI = P // TM
NJ = F // TN


def _routing_kernel(g_ref, tw_ref, ti_ref):
    logits = g_ref[...]
    cols = jax.lax.broadcasted_iota(jnp.int32, (T, E), 1)
    l1 = jnp.max(logits, axis=-1, keepdims=True)
    i1 = jnp.min(jnp.where(logits == l1, cols, E), axis=-1, keepdims=True)
    masked = jnp.where(cols == i1, -jnp.inf, logits)
    l2 = jnp.max(masked, axis=-1, keepdims=True)
    i2 = jnp.min(jnp.where(masked == l2, cols, E), axis=-1, keepdims=True)
    e2 = jnp.exp(l2 - l1)
    denom = 1.0 + e2
    tw_ref[...] = jnp.concatenate([1.0 / denom, e2 / denom], axis=1)
    ti_ref[...] = jnp.concatenate([i1, i2], axis=1).astype(jnp.int32)


def _moe_kernel(row_token_ref, tile_expert_ref, tile_valid_ref,
                x_ref, w1g_ref, w1u_ref, w2_ref, rw_ref,
                out_ref, xg, yacc):
    i = pl.program_id(0)
    j = pl.program_id(1)
    valid = tile_valid_ref[i] == 1

    @pl.when(jnp.logical_and(i == 0, j == 0))
    def _zero_out():
        out_ref[...] = jnp.zeros_like(out_ref)

    @pl.when(jnp.logical_and(j == 0, valid))
    def _gather():
        yacc[...] = jnp.zeros_like(yacc)

        def body(r, _):
            t = row_token_ref[i * TM + r]
            xg[pl.ds(r, 1), :] = x_ref[pl.ds(t, 1), :]
            return 0

        jax.lax.fori_loop(0, TM, body, 0)

    @pl.when(valid)
    def _compute():
        xx = xg[...]
        g = jnp.dot(xx, w1g_ref[0], preferred_element_type=jnp.float32)
        u = jnp.dot(xx, w1u_ref[0], preferred_element_type=jnp.float32)
        h = (g * jax.lax.logistic(g)) * u * rw_ref[...]
        yacc[...] += jnp.dot(h, w2_ref[0], preferred_element_type=jnp.float32)

    @pl.when(jnp.logical_and(j == NJ - 1, valid))
    def _scatter():
        def body(r, _):
            t = row_token_ref[i * TM + r]
            out_ref[pl.ds(t, 1), :] += yacc[pl.ds(r, 1), :]
            return 0

        jax.lax.fori_loop(0, TM, body, 0)


def kernel(hidden_states, gating_output, w1, w2):
    tw, ti = pl.pallas_call(
        _routing_kernel,
        out_shape=(
            jax.ShapeDtypeStruct((T, K), jnp.float32),
            jax.ShapeDtypeStruct((T, K), jnp.int32),
        ),
    )(gating_output)

    flat_e = ti.reshape(-1)
    flat_w = tw.reshape(-1)
    order = jnp.argsort(flat_e)
    sorted_e = flat_e[order]
    counts = jnp.zeros((E,), jnp.int32).at[flat_e].add(1)
    padded = ((counts + TM - 1) // TM) * TM
    pend = jnp.cumsum(padded)
    pstart = pend - padded
    rstart = jnp.cumsum(counts) - counts
    rank = jnp.arange(T * K, dtype=jnp.int32) - rstart[sorted_e]
    dest = pstart[sorted_e] + rank
    row_token = jnp.zeros((P,), jnp.int32).at[dest].set(order // K)
    row_w = jnp.zeros((P,), jnp.float32).at[dest].set(flat_w[order])
    tile_start = jnp.arange(NI, dtype=jnp.int32) * TM
    tile_expert = jnp.minimum(
        jnp.searchsorted(pend, tile_start, side="right"), E - 1
    ).astype(jnp.int32)
    tile_valid = (tile_start < pend[-1]).astype(jnp.int32)

    out = pl.pallas_call(
        _moe_kernel,
        grid_spec=pltpu.PrefetchScalarGridSpec(
            num_scalar_prefetch=3,
            grid=(NI, NJ),
            in_specs=[
                pl.BlockSpec((T, D), lambda i, j, *_: (0, 0)),
                pl.BlockSpec((1, D, TN), lambda i, j, rt, te, tv: (te[i], 0, j)),
                pl.BlockSpec((1, D, TN), lambda i, j, rt, te, tv: (te[i], 0, j + NJ)),
                pl.BlockSpec((1, TN, D), lambda i, j, rt, te, tv: (te[i], j, 0)),
                pl.BlockSpec((TM, 1), lambda i, j, *_: (i, 0)),
            ],
            out_specs=pl.BlockSpec((T, D), lambda i, j, *_: (0, 0)),
            scratch_shapes=[
                pltpu.VMEM((TM, D), jnp.float32),
                pltpu.VMEM((TM, D), jnp.float32),
            ],
        ),
        out_shape=jax.ShapeDtypeStruct((T, D), jnp.float32),
    )(row_token, tile_expert, tile_valid,
      hidden_states, w1, w1, w2, row_w.reshape(P, 1))
    return out


# speedup vs baseline: 1.1876x; 1.1876x over previous
import jax
import jax.numpy as jnp
from jax.experimental import pallas as pl
from jax.experimental.pallas import tpu as pltpu

E = 8
K = 2
D = 1024
F = 4096
T = 2048
TM = 128
TN = 512
TND = 512
TMO = 128
P = T * K + E * TM
NI = P // TM
NJ = F // TN
ND2 = D // TND
NTO = T // TMO


def _routing_kernel(g_ref, tw_ref, ti_ref):
    logits = g_ref[...]
    cols = jax.lax.broadcasted_iota(jnp.int32, (T, E), 1)
    l1 = jnp.max(logits, axis=-1, keepdims=True)
    i1 = jnp.min(jnp.where(logits == l1, cols, E), axis=-1, keepdims=True)
    masked = jnp.where(cols == i1, -jnp.inf, logits)
    l2 = jnp.max(masked, axis=-1, keepdims=True)
    i2 = jnp.min(jnp.where(masked == l2, cols, E), axis=-1, keepdims=True)
    e2 = jnp.exp(l2 - l1)
    denom = 1.0 + e2
    tw_ref[...] = jnp.concatenate([1.0 / denom, e2 / denom], axis=1)
    ti_ref[...] = jnp.concatenate([i1, i2], axis=1).astype(jnp.int32)


def _gemm1_kernel(row_token_ref, tile_expert_ref, tile_valid_ref,
                  x_ref, w1g_ref, w1u_ref, rw_ref, h_ref, xg):
    j = pl.program_id(0)
    i = pl.program_id(1)
    valid = tile_valid_ref[i] == 1

    @pl.when(jnp.logical_and(j == 0, valid))
    def _gather():
        def body(r, _):
            t = row_token_ref[i * TM + r]
            xg[pl.ds(i * TM + r, 1), :] = x_ref[pl.ds(t, 1), :]
            return 0

        jax.lax.fori_loop(0, TM, body, 0)

    @pl.when(valid)
    def _compute():
        xx = xg[pl.ds(i * TM, TM), :]
        g = jnp.dot(xx, w1g_ref[0], preferred_element_type=jnp.float32)
        u = jnp.dot(xx, w1u_ref[0], preferred_element_type=jnp.float32)
        h_ref[...] = (g * jax.lax.logistic(g)) * u * rw_ref[...]


def _gemm2_kernel(row_token_ref, tile_expert_ref, tile_valid_ref,
                  h_ref, w2_ref, y_ref):
    i = pl.program_id(1)
    valid = tile_valid_ref[i] == 1

    @pl.when(valid)
    def _compute():
        y_ref[...] = jnp.dot(h_ref[...], w2_ref[0],
                             preferred_element_type=jnp.float32)


def _combine_kernel(pos_ref, y_ref, out_ref):
    i = pl.program_id(0)

    def body(r, _):
        p0 = pos_ref[(i * TMO + r) * K]
        p1 = pos_ref[(i * TMO + r) * K + 1]
        out_ref[pl.ds(r, 1), :] = (y_ref[pl.ds(p0, 1), :]
                                   + y_ref[pl.ds(p1, 1), :])
        return 0

    jax.lax.fori_loop(0, TMO, body, 0)


def kernel(hidden_states, gating_output, w1, w2):
    tw, ti = pl.pallas_call(
        _routing_kernel,
        out_shape=(
            jax.ShapeDtypeStruct((T, K), jnp.float32),
            jax.ShapeDtypeStruct((T, K), jnp.int32),
        ),
    )(gating_output)

    flat_e = ti.reshape(-1)
    flat_w = tw.reshape(-1)
    order = jnp.argsort(flat_e)
    sorted_e = flat_e[order]
    counts = jnp.zeros((E,), jnp.int32).at[flat_e].add(1)
    padded = ((counts + TM - 1) // TM) * TM
    pend = jnp.cumsum(padded)
    pstart = pend - padded
    rstart = jnp.cumsum(counts) - counts
    rank = jnp.arange(T * K, dtype=jnp.int32) - rstart[sorted_e]
    dest = pstart[sorted_e] + rank
    row_token = jnp.zeros((P,), jnp.int32).at[dest].set(order // K)
    row_w = jnp.zeros((P,), jnp.float32).at[dest].set(flat_w[order])
    pos = jnp.zeros((T * K,), jnp.int32).at[order].set(dest)
    tile_start = jnp.arange(NI, dtype=jnp.int32) * TM
    tile_expert = jnp.minimum(
        jnp.searchsorted(pend, tile_start, side="right"), E - 1
    ).astype(jnp.int32)
    tile_valid = (tile_start < pend[-1]).astype(jnp.int32)

    h = pl.pallas_call(
        _gemm1_kernel,
        grid_spec=pltpu.PrefetchScalarGridSpec(
            num_scalar_prefetch=3,
            grid=(NJ, NI),
            in_specs=[
                pl.BlockSpec((T, D), lambda j, i, *_: (0, 0)),
                pl.BlockSpec((1, D, TN), lambda j, i, rt, te, tv: (te[i], 0, j)),
                pl.BlockSpec((1, D, TN), lambda j, i, rt, te, tv: (te[i], 0, j + NJ)),
                pl.BlockSpec((TM, 1), lambda j, i, *_: (i, 0)),
            ],
            out_specs=pl.BlockSpec((TM, TN), lambda j, i, *_: (i, j)),
            scratch_shapes=[pltpu.VMEM((P, D), jnp.float32)],
        ),
        out_shape=jax.ShapeDtypeStruct((P, F), jnp.float32),
    )(row_token, tile_expert, tile_valid,
      hidden_states, w1, w1, row_w.reshape(P, 1))

    y = pl.pallas_call(
        _gemm2_kernel,
        grid_spec=pltpu.PrefetchScalarGridSpec(
            num_scalar_prefetch=3,
            grid=(ND2, NI),
            in_specs=[
                pl.BlockSpec((TM, F), lambda jd, i, *_: (i, 0)),
                pl.BlockSpec((1, F, TND), lambda jd, i, rt, te, tv: (te[i], 0, jd)),
            ],
            out_specs=pl.BlockSpec((TM, TND), lambda jd, i, *_: (i, jd)),
        ),
        out_shape=jax.ShapeDtypeStruct((P, D), jnp.float32),
    )(row_token, tile_expert, tile_valid, h, w2)

    out = pl.pallas_call(
        _combine_kernel,
        grid_spec=pltpu.PrefetchScalarGridSpec(
            num_scalar_prefetch=1,
            grid=(NTO,),
            in_specs=[
                pl.BlockSpec((P, D), lambda i, *_: (0, 0)),
            ],
            out_specs=pl.BlockSpec((TMO, D), lambda i, *_: (i, 0)),
        ),
        out_shape=jax.ShapeDtypeStruct((T, D), jnp.float32),
    )(pos, y)
    return out


# speedup vs baseline: 1.3448x; 1.1323x over previous
import jax
import jax.numpy as jnp
from jax.experimental import pallas as pl
from jax.experimental.pallas import tpu as pltpu

E = 8
K = 2
D = 1024
F = 4096
T = 2048
TM = 128
TN = 512
TND = 512
TMO = 128
P = T * K + E * TM
NI = P // TM
NJ = F // TN
ND2 = D // TND
NTO = T // TMO


def _routing_kernel(g_ref, dest_ref, tw_ref, pend_ref):
    logits = g_ref[...]
    cols = jax.lax.broadcasted_iota(jnp.int32, (T, E), 1)
    l1 = jnp.max(logits, axis=-1, keepdims=True)
    i1 = jnp.min(jnp.where(logits == l1, cols, E), axis=-1, keepdims=True)
    masked = jnp.where(cols == i1, -jnp.inf, logits)
    l2 = jnp.max(masked, axis=-1, keepdims=True)
    i2 = jnp.min(jnp.where(masked == l2, cols, E), axis=-1, keepdims=True)
    e2 = jnp.exp(l2 - l1)
    denom = 1.0 + e2
    tw_ref[...] = jnp.concatenate([1.0 / denom, e2 / denom], axis=1)

    oh0 = (cols == i1).astype(jnp.float32)
    oh1 = (cols == i2).astype(jnp.float32)
    both = oh0 + oh1
    counts = jnp.sum(both, axis=0, keepdims=True)
    padded = jnp.ceil(counts / TM) * TM
    ea = jax.lax.broadcasted_iota(jnp.int32, (E, E), 0)
    eb = jax.lax.broadcasted_iota(jnp.int32, (E, E), 1)
    pend = jnp.dot(padded, (ea <= eb).astype(jnp.float32),
                   preferred_element_type=jnp.float32)
    pstart = pend - padded
    ta = jax.lax.broadcasted_iota(jnp.int32, (T, T), 0)
    tb = jax.lax.broadcasted_iota(jnp.int32, (T, T), 1)
    ltri = (tb < ta).astype(jnp.float32)
    csum = jnp.dot(ltri, both, preferred_element_type=jnp.float32)
    d0 = jnp.sum((pstart + csum) * oh0, axis=-1, keepdims=True)
    d1 = jnp.sum((pstart + csum) * oh1, axis=-1, keepdims=True)
    dest_ref[...] = jnp.concatenate([d0, d1], axis=1).astype(jnp.int32)
    pend_ref[...] = pend.astype(jnp.int32)


def _gemm1_kernel(row_token_ref, tile_expert_ref, tile_valid_ref,
                  x_ref, w1g_ref, w1u_ref, rw_ref, h_ref, xg):
    j = pl.program_id(0)
    i = pl.program_id(1)
    valid = tile_valid_ref[i] == 1

    @pl.when(jnp.logical_and(j == 0, valid))
    def _gather():
        def body(r, _):
            t = row_token_ref[i * TM + r]
            xg[pl.ds(i * TM + r, 1), :] = x_ref[pl.ds(t, 1), :]
            return 0

        jax.lax.fori_loop(0, TM, body, 0)

    @pl.when(valid)
    def _compute():
        xx = xg[pl.ds(i * TM, TM), :]
        g = jnp.dot(xx, w1g_ref[0], preferred_element_type=jnp.float32)
        u = jnp.dot(xx, w1u_ref[0], preferred_element_type=jnp.float32)
        h_ref[...] = (g * jax.lax.logistic(g)) * u * rw_ref[...]


def _gemm2_kernel(row_token_ref, tile_expert_ref, tile_valid_ref,
                  h_ref, w2_ref, y_ref):
    i = pl.program_id(1)
    valid = tile_valid_ref[i] == 1

    @pl.when(valid)
    def _compute():
        y_ref[...] = jnp.dot(h_ref[...], w2_ref[0],
                             preferred_element_type=jnp.float32)


def _combine_kernel(pos_ref, y_ref, out_ref):
    i = pl.program_id(0)

    def body(r, _):
        p0 = pos_ref[(i * TMO + r) * K]
        p1 = pos_ref[(i * TMO + r) * K + 1]
        out_ref[pl.ds(r, 1), :] = (y_ref[pl.ds(p0, 1), :]
                                   + y_ref[pl.ds(p1, 1), :])
        return 0

    jax.lax.fori_loop(0, TMO, body, 0)


def kernel(hidden_states, gating_output, w1, w2):
    dest, tw, pend = pl.pallas_call(
        _routing_kernel,
        out_shape=(
            jax.ShapeDtypeStruct((T, K), jnp.int32),
            jax.ShapeDtypeStruct((T, K), jnp.float32),
            jax.ShapeDtypeStruct((1, E), jnp.int32),
        ),
    )(gating_output)

    pos = dest.reshape(-1)
    row_token = jnp.zeros((P,), jnp.int32).at[pos].set(
        jnp.arange(T * K, dtype=jnp.int32) // K)
    row_w = jnp.zeros((P,), jnp.float32).at[pos].set(tw.reshape(-1))
    pend1 = pend.reshape(-1)
    tile_start = jnp.arange(NI, dtype=jnp.int32) * TM
    tile_expert = jnp.minimum(
        jnp.sum((tile_start[:, None] >= pend1[None, :]).astype(jnp.int32),
                axis=1), E - 1).astype(jnp.int32)
    tile_valid = (tile_start < pend1[-1]).astype(jnp.int32)

    h = pl.pallas_call(
        _gemm1_kernel,
        grid_spec=pltpu.PrefetchScalarGridSpec(
            num_scalar_prefetch=3,
            grid=(NJ, NI),
            in_specs=[
                pl.BlockSpec((T, D), lambda j, i, *_: (0, 0)),
                pl.BlockSpec((1, D, TN), lambda j, i, rt, te, tv: (te[i], 0, j)),
                pl.BlockSpec((1, D, TN), lambda j, i, rt, te, tv: (te[i], 0, j + NJ)),
                pl.BlockSpec((TM, 1), lambda j, i, *_: (i, 0)),
            ],
            out_specs=pl.BlockSpec((TM, TN), lambda j, i, *_: (i, j)),
            scratch_shapes=[pltpu.VMEM((P, D), jnp.float32)],
        ),
        out_shape=jax.ShapeDtypeStruct((P, F), jnp.float32),
    )(row_token, tile_expert, tile_valid,
      hidden_states, w1, w1, row_w.reshape(P, 1))

    y = pl.pallas_call(
        _gemm2_kernel,
        grid_spec=pltpu.PrefetchScalarGridSpec(
            num_scalar_prefetch=3,
            grid=(ND2, NI),
            in_specs=[
                pl.BlockSpec((TM, F), lambda jd, i, *_: (i, 0)),
                pl.BlockSpec((1, F, TND), lambda jd, i, rt, te, tv: (te[i], 0, jd)),
            ],
            out_specs=pl.BlockSpec((TM, TND), lambda jd, i, *_: (i, jd)),
        ),
        out_shape=jax.ShapeDtypeStruct((P, D), jnp.float32),
    )(row_token, tile_expert, tile_valid, h, w2)

    out = pl.pallas_call(
        _combine_kernel,
        grid_spec=pltpu.PrefetchScalarGridSpec(
            num_scalar_prefetch=1,
            grid=(NTO,),
            in_specs=[
                pl.BlockSpec((P, D), lambda i, *_: (0, 0)),
            ],
            out_specs=pl.BlockSpec((TMO, D), lambda i, *_: (i, 0)),
        ),
        out_shape=jax.ShapeDtypeStruct((T, D), jnp.float32),
    )(pos, y)
    return out
